# baseline (speedup 1.0000x reference)
import jax
import jax.numpy as jnp
from jax.experimental import pallas as pl
from jax.experimental.pallas import tpu as pltpu
from jax.experimental.pallas import tpu_sc as plsc

_R = 16


def kernel(input_ids, embedding_table):
    batch, seq = input_ids.shape
    dim = embedding_table.shape[1]

    mesh = plsc.VectorSubcoreMesh(core_axis_name="core",
                                  subcore_axis_name="subcore")

    @pl.kernel(out_type=jax.ShapeDtypeStruct((batch, seq, dim),
                                             embedding_table.dtype),
               mesh=mesh,
               scratch_types=[pltpu.SemaphoreType.DMA],
               compiler_params=pltpu.CompilerParams(use_tc_tiling_on_sc=False))
    def gather_kernel(table_hbm, i_hbm, o_hbm, sem):
        def body(i_vmem, o_vmem):
            copies = [
                pltpu.async_copy(table_hbm.at[i_vmem.at[j]],
                                 o_vmem.at[j], sem)
                for j in range(_R)
            ]
            for c in copies:
                c.wait()

        pltpu.emit_pipeline(
            body,
            grid=(batch // _R,),
            in_specs=[pl.BlockSpec((_R, seq), lambda i: (i, 0))],
            out_specs=[pl.BlockSpec((_R, seq, dim), lambda i: (i, 0, 0))],
            core_axis_name=("core", "subcore"),
            dimension_semantics=(pltpu.PARALLEL,),
        )(i_hbm, o_hbm)

    embeddings = gather_kernel(embedding_table, input_ids)
    attention_mask = jnp.ones((batch, seq), dtype=jnp.int32)
    return (embeddings, attention_mask)
